# baseline (speedup 1.0000x reference)
import functools

import jax
import jax.numpy as jnp
import numpy as np
from jax import lax
from jax.experimental import pallas as pl
from jax.experimental.pallas import tpu as pltpu
from jax.experimental.pallas import tpu_sc as plsc

VOCAB = 100000
HIDDEN = 128
HALF = HIDDEN // 2
MAX_POS = 512
B = 4096
L = 200
EPS = 1e-12

NC = 2
NS = 16
NW = NC * NS

N_XCHUNK = 4
B_C = B // N_XCHUNK
ROWS_C = B_C * L
ROWS_PER_W = ROWS_C // NW
CHUNK = 400
N_CHUNKS = ROWS_PER_W // CHUNK

RB = 64
RBH = RB // 2
TC_STEPS = B_C // RB
TOK_BLK = RB * L
HTOK = TOK_BLK // 2

_PERM = np.empty(HIDDEN, np.int32)
_PERM[0::2] = np.arange(HALF)
_PERM[1::2] = np.arange(HALF, HIDDEN)


def _sc_gather(table, flat_ids):
    mesh = plsc.VectorSubcoreMesh(core_axis_name="c", subcore_axis_name="s")

    @functools.partial(
        pl.kernel,
        out_type=jax.ShapeDtypeStruct((ROWS_C, HALF), jnp.int32),
        mesh=mesh,
        scratch_types=[
            pltpu.VMEM((CHUNK,), jnp.int32),
            pltpu.VMEM((CHUNK,), jnp.int32),
            pltpu.VMEM((CHUNK, HALF), jnp.int32),
            pltpu.VMEM((CHUNK, HALF), jnp.int32),
            pltpu.SemaphoreType.DMA,
            pltpu.SemaphoreType.DMA,
            pltpu.SemaphoreType.DMA,
            pltpu.SemaphoreType.DMA,
        ],
        compiler_params=pltpu.CompilerParams(use_tc_tiling_on_sc=False),
    )
    def gather_kernel(table_hbm, ids_hbm, out_hbm,
                      idx0, idx1, rows0, rows1, g0, g1, w0, w1):
        wid = lax.axis_index("s") * NC + lax.axis_index("c")
        base = wid * ROWS_PER_W
        idx = (idx0, idx1)
        rows = (rows0, rows1)
        gsem = (g0, g1)
        wsem = (w0, w1)

        pltpu.sync_copy(ids_hbm.at[pl.ds(base, CHUNK)], idx0)
        pltpu.async_copy(table_hbm.at[idx0], rows0, g0)

        @pl.loop(0, N_CHUNKS // 2)
        def _(p):
            for b in (0, 1):
                c = 2 * p + b
                nb = 1 - b
                @pl.when(c + 1 < N_CHUNKS)
                def _():
                    off_n = base + (c + 1) * CHUNK
                    pltpu.sync_copy(ids_hbm.at[pl.ds(off_n, CHUNK)], idx[nb])
                    @pl.when(c >= 1)
                    def _():
                        pltpu.make_async_copy(
                            rows[nb], out_hbm.at[pl.ds(base, CHUNK)],
                            wsem[nb]).wait()
                    pltpu.async_copy(table_hbm.at[idx[nb]], rows[nb], gsem[nb])

                pltpu.make_async_copy(
                    table_hbm.at[idx[b]], rows[b], gsem[b]).wait()
                pltpu.async_copy(
                    rows[b], out_hbm.at[pl.ds(base + c * CHUNK, CHUNK)],
                    wsem[b])

        pltpu.make_async_copy(
            rows0, out_hbm.at[pl.ds(base, CHUNK)], w0).wait()
        pltpu.make_async_copy(
            rows1, out_hbm.at[pl.ds(base, CHUNK)], w1).wait()

    return gather_kernel(table, flat_ids)


def _ln_norm(x, pos, gamma, beta):
    x = x + pos[None]
    mean = jnp.mean(x, axis=-1, keepdims=True)
    var = jnp.mean(jnp.square(x - mean), axis=-1, keepdims=True)
    return (x - mean) * lax.rsqrt(var + EPS) * gamma + beta


def _ln_math(x_ref, pos_ref, g_ref, b_ref, o_ref):
    xi = x_ref[0]
    lo = lax.bitcast_convert_type(xi << 16, jnp.float32)
    hi = lax.bitcast_convert_type(xi & jnp.int32(-65536), jnp.float32)
    xa = jnp.concatenate([lo[:, :HALF], hi[:, :HALF]], axis=-1)
    xb = jnp.concatenate([lo[:, HALF:], hi[:, HALF:]], axis=-1)
    pos = pos_ref[...]
    gamma = g_ref[...]
    beta = b_ref[...]
    na = _ln_norm(xa.reshape(RBH, L, HIDDEN), pos, gamma, beta)
    nb = _ln_norm(xb.reshape(RBH, L, HIDDEN), pos, gamma, beta)
    o_ref[:RBH] = na
    o_ref[RBH:] = nb


def _ln_body(prev_ref, x_ref, pos_ref, g_ref, b_ref, o_ref):
    del prev_ref
    _ln_math(x_ref, pos_ref, g_ref, b_ref, o_ref)


def _tc_ln_chunk(prev_out, gathered, pos, gamma, beta, chunk):
    base = chunk * TC_STEPS
    return pl.pallas_call(
        _ln_body,
        grid=(TC_STEPS,),
        in_specs=[
            pl.BlockSpec((8, 8, HIDDEN), lambda i: (0, 0, 0)),
            pl.BlockSpec((1, HTOK, HIDDEN), lambda i: (i, 0, 0)),
            pl.BlockSpec((L, HIDDEN), lambda i: (0, 0)),
            pl.BlockSpec((HIDDEN,), lambda i: (0,)),
            pl.BlockSpec((HIDDEN,), lambda i: (0,)),
        ],
        out_specs=pl.BlockSpec((RB, L, HIDDEN), lambda i: (base + i, 0, 0)),
        out_shape=jax.ShapeDtypeStruct((B, L, HIDDEN), jnp.float32),
        input_output_aliases={0: 0},
    )(prev_out, gathered, pos, gamma, beta)


def _tc_ln_first(gathered, pos, gamma, beta):
    return pl.pallas_call(
        _ln_math,
        grid=(TC_STEPS,),
        in_specs=[
            pl.BlockSpec((1, HTOK, HIDDEN), lambda i: (i, 0, 0)),
            pl.BlockSpec((L, HIDDEN), lambda i: (0, 0)),
            pl.BlockSpec((HIDDEN,), lambda i: (0,)),
            pl.BlockSpec((HIDDEN,), lambda i: (0,)),
        ],
        out_specs=pl.BlockSpec((RB, L, HIDDEN), lambda i: (i, 0, 0)),
        out_shape=jax.ShapeDtypeStruct((B, L, HIDDEN), jnp.float32),
    )(gathered, pos, gamma, beta)


def kernel(input_ids, word_embeddings, position_embeddings, ln_gamma, ln_beta):
    pos = position_embeddings[:L]

    tbl = word_embeddings[:, _PERM].astype(jnp.bfloat16)
    tbl_packed = lax.bitcast_convert_type(
        tbl.reshape(VOCAB, HALF, 2), jnp.int32)

    ids_perm = (input_ids.reshape(B // RB, 2, HTOK)
                .transpose(0, 2, 1).reshape(-1))

    gathered = [
        _sc_gather(tbl_packed, ids_perm[c * ROWS_C:(c + 1) * ROWS_C])
        .reshape(TC_STEPS, HTOK, HIDDEN)
        for c in range(N_XCHUNK)
    ]
    out = _tc_ln_first(gathered[0], pos, ln_gamma, ln_beta)
    for c in range(1, N_XCHUNK):
        out = _tc_ln_chunk(out, gathered[c], pos, ln_gamma, ln_beta, c)
    return out


# speedup vs baseline: 1.2904x; 1.2904x over previous
import functools

import jax
import jax.numpy as jnp
import numpy as np
from jax import lax
from jax.experimental import pallas as pl
from jax.experimental.pallas import tpu as pltpu
from jax.experimental.pallas import tpu_sc as plsc

VOCAB = 100000
HIDDEN = 128
HALF = HIDDEN // 2
MAX_POS = 512
B = 4096
L = 200
EPS = 1e-12

NC = 2
NS = 16
NW = NC * NS

N_XCHUNK = 4
B_C = B // N_XCHUNK
ROWS_C = B_C * L
ROWS_PER_W = ROWS_C // NW
CHUNK = 400
N_CHUNKS = ROWS_PER_W // CHUNK

RB = 64
RBH = RB // 2
TC_STEPS = B_C // RB
TOK_BLK = RB * L
HTOK = TOK_BLK // 2

_PERM = np.empty(HIDDEN, np.int32)
_PERM[0::2] = np.arange(HALF)
_PERM[1::2] = np.arange(HALF, HIDDEN)


CHUNKP = 400
PAIRS_PER_W = (ROWS_C // 2) // NW
N_IT = PAIRS_PER_W // CHUNKP


def _sc_gather(table, flat_ids):
    mesh = plsc.VectorSubcoreMesh(core_axis_name="c", subcore_axis_name="s")

    @functools.partial(
        pl.kernel,
        out_type=jax.ShapeDtypeStruct((TC_STEPS, HTOK, HIDDEN), jnp.int32),
        mesh=mesh,
        scratch_types=[
            pltpu.VMEM((CHUNKP,), jnp.int32),
            pltpu.VMEM((CHUNKP,), jnp.int32),
            pltpu.VMEM((CHUNKP,), jnp.int32),
            pltpu.VMEM((CHUNKP,), jnp.int32),
            pltpu.VMEM((CHUNKP, HALF), jnp.int32),
            pltpu.VMEM((CHUNKP, HALF), jnp.int32),
            pltpu.VMEM((CHUNKP, HALF), jnp.int32),
            pltpu.VMEM((CHUNKP, HALF), jnp.int32),
            pltpu.SemaphoreType.DMA,
            pltpu.SemaphoreType.DMA,
            pltpu.SemaphoreType.DMA,
            pltpu.SemaphoreType.DMA,
            pltpu.SemaphoreType.DMA,
            pltpu.SemaphoreType.DMA,
            pltpu.SemaphoreType.DMA,
            pltpu.SemaphoreType.DMA,
        ],
        compiler_params=pltpu.CompilerParams(use_tc_tiling_on_sc=False),
    )
    def gather_kernel(table_hbm, ids_hbm, out_hbm,
                      ia0, ia1, ib0, ib1, ra0, ra1, rb0, rb1,
                      ga0, ga1, gb0, gb1, wa0, wa1, wb0, wb1):
        wid = lax.axis_index("s") * NC + lax.axis_index("c")
        blk = wid // 2
        half = wid % 2
        tok0 = blk * TOK_BLK + half * PAIRS_PER_W
        q0 = half * PAIRS_PER_W
        ia = (ia0, ia1)
        ib = (ib0, ib1)
        ra = (ra0, ra1)
        rb = (rb0, rb1)
        gsa = (ga0, ga1)
        gsb = (gb0, gb1)
        wsa = (wa0, wa1)
        wsb = (wb0, wb1)

        def fetch_idx(c, b):
            off = c * CHUNKP
            pltpu.sync_copy(ids_hbm.at[pl.ds(tok0 + off, CHUNKP)], ia[b])
            pltpu.sync_copy(ids_hbm.at[pl.ds(tok0 + HTOK + off, CHUNKP)],
                            ib[b])

        def start_gathers(b):
            pltpu.async_copy(table_hbm.at[ia[b]], ra[b], gsa[b])
            pltpu.async_copy(table_hbm.at[ib[b]], rb[b], gsb[b])

        def wait_gathers(b):
            pltpu.make_async_copy(table_hbm.at[ia[b]], ra[b], gsa[b]).wait()
            pltpu.make_async_copy(table_hbm.at[ib[b]], rb[b], gsb[b]).wait()

        def start_writebacks(c, b):
            q = q0 + c * CHUNKP
            pltpu.async_copy(
                ra[b], out_hbm.at[blk, pl.ds(q, CHUNKP), pl.ds(0, HALF)],
                wsa[b])
            pltpu.async_copy(
                rb[b], out_hbm.at[blk, pl.ds(q, CHUNKP), pl.ds(HALF, HALF)],
                wsb[b])

        def wait_writebacks(b):
            pltpu.make_async_copy(
                ra[b], out_hbm.at[blk, pl.ds(q0, CHUNKP), pl.ds(0, HALF)],
                wsa[b]).wait()
            pltpu.make_async_copy(
                rb[b], out_hbm.at[blk, pl.ds(q0, CHUNKP), pl.ds(HALF, HALF)],
                wsb[b]).wait()

        fetch_idx(0, 0)
        start_gathers(0)

        @pl.loop(0, N_IT // 2)
        def _(p):
            for b in (0, 1):
                c = 2 * p + b
                nb = 1 - b

                @pl.when(c + 1 < N_IT)
                def _():
                    fetch_idx(c + 1, nb)
                    @pl.when(c >= 1)
                    def _():
                        wait_writebacks(nb)
                    start_gathers(nb)

                wait_gathers(b)
                start_writebacks(c, b)

        wait_writebacks(0)
        wait_writebacks(1)

    return gather_kernel(table, flat_ids)


def _ln_norm(x, pos, gamma, beta):
    x = x + pos[None]
    mean = jnp.mean(x, axis=-1, keepdims=True)
    var = jnp.mean(jnp.square(x - mean), axis=-1, keepdims=True)
    return (x - mean) * lax.rsqrt(var + EPS) * gamma + beta


def _ln_math(x_ref, pos_ref, g_ref, b_ref, o_ref):
    xi = x_ref[0]
    lo = lax.bitcast_convert_type(xi << 16, jnp.float32)
    hi = lax.bitcast_convert_type(xi & jnp.int32(-65536), jnp.float32)
    xa = jnp.concatenate([lo[:, :HALF], hi[:, :HALF]], axis=-1)
    xb = jnp.concatenate([lo[:, HALF:], hi[:, HALF:]], axis=-1)
    pos = pos_ref[...]
    gamma = g_ref[...]
    beta = b_ref[...]
    na = _ln_norm(xa.reshape(RBH, L, HIDDEN), pos, gamma, beta)
    nb = _ln_norm(xb.reshape(RBH, L, HIDDEN), pos, gamma, beta)
    o_ref[:RBH] = na
    o_ref[RBH:] = nb


def _ln_body(prev_ref, x_ref, pos_ref, g_ref, b_ref, o_ref):
    del prev_ref
    _ln_math(x_ref, pos_ref, g_ref, b_ref, o_ref)


def _tc_ln_chunk(prev_out, gathered, pos, gamma, beta, chunk):
    base = chunk * TC_STEPS
    return pl.pallas_call(
        _ln_body,
        grid=(TC_STEPS,),
        in_specs=[
            pl.BlockSpec((8, 8, HIDDEN), lambda i: (0, 0, 0)),
            pl.BlockSpec((1, HTOK, HIDDEN), lambda i: (i, 0, 0)),
            pl.BlockSpec((L, HIDDEN), lambda i: (0, 0)),
            pl.BlockSpec((HIDDEN,), lambda i: (0,)),
            pl.BlockSpec((HIDDEN,), lambda i: (0,)),
        ],
        out_specs=pl.BlockSpec((RB, L, HIDDEN), lambda i: (base + i, 0, 0)),
        out_shape=jax.ShapeDtypeStruct((B, L, HIDDEN), jnp.float32),
        input_output_aliases={0: 0},
    )(prev_out, gathered, pos, gamma, beta)


def _tc_ln_first(gathered, pos, gamma, beta):
    return pl.pallas_call(
        _ln_math,
        grid=(TC_STEPS,),
        in_specs=[
            pl.BlockSpec((1, HTOK, HIDDEN), lambda i: (i, 0, 0)),
            pl.BlockSpec((L, HIDDEN), lambda i: (0, 0)),
            pl.BlockSpec((HIDDEN,), lambda i: (0,)),
            pl.BlockSpec((HIDDEN,), lambda i: (0,)),
        ],
        out_specs=pl.BlockSpec((RB, L, HIDDEN), lambda i: (i, 0, 0)),
        out_shape=jax.ShapeDtypeStruct((B, L, HIDDEN), jnp.float32),
    )(gathered, pos, gamma, beta)


def kernel(input_ids, word_embeddings, position_embeddings, ln_gamma, ln_beta):
    pos = position_embeddings[:L]

    tbl = word_embeddings[:, _PERM].astype(jnp.bfloat16)
    tbl_packed = lax.bitcast_convert_type(
        tbl.reshape(VOCAB, HALF, 2), jnp.int32)

    flat_ids = input_ids.reshape(-1)
    gathered = [
        _sc_gather(tbl_packed, flat_ids[c * ROWS_C:(c + 1) * ROWS_C])
        for c in range(N_XCHUNK)
    ]
    out = _tc_ln_first(gathered[0], pos, ln_gamma, ln_beta)
    for c in range(1, N_XCHUNK):
        out = _tc_ln_chunk(out, gathered[c], pos, ln_gamma, ln_beta, c)
    return out


# speedup vs baseline: 2.1431x; 1.6609x over previous
import functools

import jax
import jax.numpy as jnp
from jax import lax
from jax.experimental import pallas as pl
from jax.experimental.pallas import tpu as pltpu
from jax.experimental.pallas import tpu_sc as plsc

VOCAB = 100000
HIDDEN = 128
MAX_POS = 512
B = 4096
L = 200
EPS = 1e-12

NC = 2
NS = 16
NW = NC * NS

CHUNK = 400
RB = 64

CHUNK_BATCHES = (512, 1536, 1536, 512)
assert sum(CHUNK_BATCHES) == B


def _sc_gather(table, flat_ids, bc):
    rows_c = bc * L
    rows_per_w = rows_c // NW
    n_rounds = rows_per_w // CHUNK
    mesh = plsc.VectorSubcoreMesh(core_axis_name="c", subcore_axis_name="s")

    @functools.partial(
        pl.kernel,
        out_type=jax.ShapeDtypeStruct((rows_c, HIDDEN), jnp.float32),
        mesh=mesh,
        scratch_types=[
            pltpu.VMEM((CHUNK,), jnp.int32),
            pltpu.VMEM((CHUNK,), jnp.int32),
            pltpu.VMEM((CHUNK, HIDDEN), jnp.float32),
            pltpu.VMEM((CHUNK, HIDDEN), jnp.float32),
            pltpu.SemaphoreType.DMA,
            pltpu.SemaphoreType.DMA,
            pltpu.SemaphoreType.DMA,
            pltpu.SemaphoreType.DMA,
        ],
    )
    def gather_kernel(table_hbm, ids_hbm, out_hbm,
                      idx0, idx1, rows0, rows1, g0, g1, w0, w1):
        wid = lax.axis_index("s") * NC + lax.axis_index("c")
        base = wid * rows_per_w
        idx = (idx0, idx1)
        rows = (rows0, rows1)
        gsem = (g0, g1)
        wsem = (w0, w1)

        pltpu.sync_copy(ids_hbm.at[pl.ds(base, CHUNK)], idx0)
        pltpu.async_copy(table_hbm.at[idx0], rows0, g0)

        @pl.loop(0, n_rounds // 2)
        def _(p):
            for b in (0, 1):
                c = 2 * p + b
                nb = 1 - b
                @pl.when(c + 1 < n_rounds)
                def _():
                    off_n = base + (c + 1) * CHUNK
                    pltpu.sync_copy(ids_hbm.at[pl.ds(off_n, CHUNK)], idx[nb])
                    @pl.when(c >= 1)
                    def _():
                        pltpu.make_async_copy(
                            rows[nb], out_hbm.at[pl.ds(base, CHUNK)],
                            wsem[nb]).wait()
                    pltpu.async_copy(table_hbm.at[idx[nb]], rows[nb], gsem[nb])

                pltpu.make_async_copy(
                    table_hbm.at[idx[b]], rows[b], gsem[b]).wait()
                pltpu.async_copy(
                    rows[b], out_hbm.at[pl.ds(base + c * CHUNK, CHUNK)],
                    wsem[b])

        pltpu.make_async_copy(
            rows0, out_hbm.at[pl.ds(base, CHUNK)], w0).wait()
        pltpu.make_async_copy(
            rows1, out_hbm.at[pl.ds(base, CHUNK)], w1).wait()

    return gather_kernel(table, flat_ids)


def _ln_math(x_ref, pos_ref, g_ref, b_ref, o_ref):
    x = x_ref[...] + pos_ref[...][None, :, :]
    mean = jnp.mean(x, axis=-1, keepdims=True)
    var = jnp.mean(jnp.square(x - mean), axis=-1, keepdims=True)
    normed = (x - mean) * lax.rsqrt(var + EPS)
    o_ref[...] = normed * g_ref[...] + b_ref[...]


def _ln_body(prev_ref, x_ref, pos_ref, g_ref, b_ref, o_ref):
    del prev_ref
    _ln_math(x_ref, pos_ref, g_ref, b_ref, o_ref)


def _tc_ln_chunk(prev_out, gathered, pos, gamma, beta, base_blk, steps):
    return pl.pallas_call(
        _ln_body,
        grid=(steps,),
        in_specs=[
            pl.BlockSpec((8, 8, HIDDEN), lambda i: (0, 0, 0)),
            pl.BlockSpec((RB, L, HIDDEN), lambda i: (i, 0, 0)),
            pl.BlockSpec((L, HIDDEN), lambda i: (0, 0)),
            pl.BlockSpec((HIDDEN,), lambda i: (0,)),
            pl.BlockSpec((HIDDEN,), lambda i: (0,)),
        ],
        out_specs=pl.BlockSpec(
            (RB, L, HIDDEN), lambda i: (base_blk + i, 0, 0)),
        out_shape=jax.ShapeDtypeStruct((B, L, HIDDEN), jnp.float32),
        input_output_aliases={0: 0},
    )(prev_out, gathered, pos, gamma, beta)


def _tc_ln_first(gathered, pos, gamma, beta, steps):
    return pl.pallas_call(
        _ln_math,
        grid=(steps,),
        in_specs=[
            pl.BlockSpec((RB, L, HIDDEN), lambda i: (i, 0, 0)),
            pl.BlockSpec((L, HIDDEN), lambda i: (0, 0)),
            pl.BlockSpec((HIDDEN,), lambda i: (0,)),
            pl.BlockSpec((HIDDEN,), lambda i: (0,)),
        ],
        out_specs=pl.BlockSpec((RB, L, HIDDEN), lambda i: (i, 0, 0)),
        out_shape=jax.ShapeDtypeStruct((B, L, HIDDEN), jnp.float32),
    )(gathered, pos, gamma, beta)


def kernel(input_ids, word_embeddings, position_embeddings, ln_gamma, ln_beta):
    flat_ids = input_ids.reshape(-1)
    pos = position_embeddings[:L]

    gathered = []
    b0 = 0
    for bc in CHUNK_BATCHES:
        rows_c = bc * L
        g = _sc_gather(word_embeddings,
                       flat_ids[b0 * L:b0 * L + rows_c], bc)
        gathered.append(g.reshape(bc, L, HIDDEN))
        b0 += bc

    out = _tc_ln_first(gathered[0], pos, ln_gamma, ln_beta,
                       CHUNK_BATCHES[0] // RB)
    base_blk = CHUNK_BATCHES[0] // RB
    for c in range(1, len(CHUNK_BATCHES)):
        out = _tc_ln_chunk(out, gathered[c], pos, ln_gamma, ln_beta,
                           base_blk, CHUNK_BATCHES[c] // RB)
        base_blk += CHUNK_BATCHES[c] // RB
    return out


# speedup vs baseline: 2.1560x; 1.0060x over previous
import functools

import jax
import jax.numpy as jnp
from jax import lax
from jax.experimental import pallas as pl
from jax.experimental.pallas import tpu as pltpu
from jax.experimental.pallas import tpu_sc as plsc

VOCAB = 100000
HIDDEN = 128
MAX_POS = 512
B = 4096
L = 200
EPS = 1e-12

NC = 2
NS = 16
NW = NC * NS

N_XCHUNK = 4
B_C = B // N_XCHUNK
ROWS_C = B_C * L
ROWS_PER_W = ROWS_C // NW
CHUNK = 400
N_CHUNKS = ROWS_PER_W // CHUNK

RB = 64
TC_STEPS = B_C // RB


def _sc_gather(table, flat_ids):
    mesh = plsc.VectorSubcoreMesh(core_axis_name="c", subcore_axis_name="s")

    @functools.partial(
        pl.kernel,
        out_type=jax.ShapeDtypeStruct((ROWS_C, HIDDEN), jnp.float32),
        mesh=mesh,
        scratch_types=[
            pltpu.VMEM((CHUNK,), jnp.int32),
            pltpu.VMEM((CHUNK,), jnp.int32),
            pltpu.VMEM((CHUNK, HIDDEN), jnp.float32),
            pltpu.VMEM((CHUNK, HIDDEN), jnp.float32),
            pltpu.SemaphoreType.DMA,
            pltpu.SemaphoreType.DMA,
            pltpu.SemaphoreType.DMA,
            pltpu.SemaphoreType.DMA,
        ],
    )
    def gather_kernel(table_hbm, ids_hbm, out_hbm,
                      idx0, idx1, rows0, rows1, g0, g1, w0, w1):
        wid = lax.axis_index("s") * NC + lax.axis_index("c")
        base = wid * ROWS_PER_W
        idx = (idx0, idx1)
        rows = (rows0, rows1)
        gsem = (g0, g1)
        wsem = (w0, w1)

        pltpu.sync_copy(ids_hbm.at[pl.ds(base, CHUNK)], idx0)
        pltpu.async_copy(table_hbm.at[idx0], rows0, g0)

        @pl.loop(0, N_CHUNKS // 2)
        def _(p):
            for b in (0, 1):
                c = 2 * p + b
                nb = 1 - b
                @pl.when(c + 1 < N_CHUNKS)
                def _():
                    off_n = base + (c + 1) * CHUNK
                    pltpu.sync_copy(ids_hbm.at[pl.ds(off_n, CHUNK)], idx[nb])
                    @pl.when(c >= 1)
                    def _():
                        pltpu.make_async_copy(
                            rows[nb], out_hbm.at[pl.ds(base, CHUNK)],
                            wsem[nb]).wait()
                    pltpu.async_copy(table_hbm.at[idx[nb]], rows[nb], gsem[nb])

                pltpu.make_async_copy(
                    table_hbm.at[idx[b]], rows[b], gsem[b]).wait()
                pltpu.async_copy(
                    rows[b], out_hbm.at[pl.ds(base + c * CHUNK, CHUNK)],
                    wsem[b])

        pltpu.make_async_copy(
            rows0, out_hbm.at[pl.ds(base, CHUNK)], w0).wait()
        pltpu.make_async_copy(
            rows1, out_hbm.at[pl.ds(base, CHUNK)], w1).wait()

    return gather_kernel(table, flat_ids)


def _ln_math(x_ref, pos_ref, g_ref, b_ref, o_ref):
    x = x_ref[...] + pos_ref[...][None, :, :]
    mean = jnp.mean(x, axis=-1, keepdims=True)
    var = jnp.mean(jnp.square(x - mean), axis=-1, keepdims=True)
    normed = (x - mean) * lax.rsqrt(var + EPS)
    o_ref[...] = normed * g_ref[...] + b_ref[...]


def _ln_body(prev_ref, x_ref, pos_ref, g_ref, b_ref, o_ref):
    del prev_ref
    _ln_math(x_ref, pos_ref, g_ref, b_ref, o_ref)


def _tc_ln_chunk(prev_out, gathered, pos, gamma, beta, chunk):
    base = chunk * TC_STEPS
    return pl.pallas_call(
        _ln_body,
        grid=(TC_STEPS,),
        in_specs=[
            pl.BlockSpec((8, 8, HIDDEN), lambda i: (0, 0, 0)),
            pl.BlockSpec((RB, L, HIDDEN), lambda i: (i, 0, 0)),
            pl.BlockSpec((L, HIDDEN), lambda i: (0, 0)),
            pl.BlockSpec((HIDDEN,), lambda i: (0,)),
            pl.BlockSpec((HIDDEN,), lambda i: (0,)),
        ],
        out_specs=pl.BlockSpec((RB, L, HIDDEN), lambda i: (base + i, 0, 0)),
        out_shape=jax.ShapeDtypeStruct((B, L, HIDDEN), jnp.float32),
        input_output_aliases={0: 0},
    )(prev_out, gathered, pos, gamma, beta)


def _tc_ln_first(gathered, pos, gamma, beta):
    return pl.pallas_call(
        _ln_math,
        grid=(TC_STEPS,),
        in_specs=[
            pl.BlockSpec((RB, L, HIDDEN), lambda i: (i, 0, 0)),
            pl.BlockSpec((L, HIDDEN), lambda i: (0, 0)),
            pl.BlockSpec((HIDDEN,), lambda i: (0,)),
            pl.BlockSpec((HIDDEN,), lambda i: (0,)),
        ],
        out_specs=pl.BlockSpec((RB, L, HIDDEN), lambda i: (i, 0, 0)),
        out_shape=jax.ShapeDtypeStruct((B, L, HIDDEN), jnp.float32),
    )(gathered, pos, gamma, beta)


def kernel(input_ids, word_embeddings, position_embeddings, ln_gamma, ln_beta):
    flat_ids = input_ids.reshape(-1)
    pos = position_embeddings[:L]

    gathered = [
        _sc_gather(word_embeddings, flat_ids[c * ROWS_C:(c + 1) * ROWS_C])
        .reshape(B_C, L, HIDDEN)
        for c in range(N_XCHUNK)
    ]
    out = _tc_ln_first(gathered[0], pos, ln_gamma, ln_beta)
    for c in range(1, N_XCHUNK):
        out = _tc_ln_chunk(out, gathered[c], pos, ln_gamma, ln_beta, c)
    return out
